# Initial kernel scaffold; baseline (speedup 1.0000x reference)
import jax
import jax.numpy as jnp
from jax.experimental import pallas as pl


def kernel(edge_index, edge_weight, user_table, item_table):
    raise NotImplementedError("write your pallas kernel here")


# speedup vs baseline: 3.9487x; 3.9487x over previous
import functools

import jax
import jax.numpy as jnp
from jax import lax
from jax.experimental import pallas as pl
from jax.experimental.pallas import tpu as pltpu
from jax.experimental.pallas import tpu_sc as plsc

N_USERS = 25000
N_ITEMS = 25000
N = N_USERS + N_ITEMS
E = 800000
D = 64
HALF = 32
N_LAYERS = 3

NC = 2
NS = 16
N_PAD = 50048
BATCH = 128
CHUNK_B = 4
EB_PER_TILE = 392
E_PAD = NS * EB_PER_TILE * BATCH
MEAN_ROWS = 92


def _build(n_pad, ns, nc, eb_per_tile, chunk_b, mean_rows, interpret=False):
    rows_per_tile = n_pad // ns
    n_chunks = eb_per_tile // chunk_b
    mean_chunks = rows_per_tile // mean_rows
    r_edge = ns * eb_per_tile

    def body(*refs):
        (src_hbm, dst_hbm, w_hbm, x0_hbm, z_hbm,
         out_hbm, x1_hbm, x2_hbm,
         acc, rv, src_v, w_v) = refs[:12]
        dst_vs = refs[12:12 + chunk_b]
        b0, b1, b2, b3, sem = refs[12 + chunk_b:]
        c = lax.axis_index("c")
        s = lax.axis_index("s")
        row0 = pl.multiple_of(s * rows_per_tile, 8)
        erow0 = pl.multiple_of(s * eb_per_tile, 8)
        hrow0 = pl.multiple_of(c * n_pad + s * rows_per_tile, 8)

        def edges_phase(xprev):
            def chunk_body(i, carry):
                r0 = erow0 + i * chunk_b
                pltpu.sync_copy(src_hbm.at[c, pl.ds(r0, chunk_b)], src_v)
                pltpu.sync_copy(w_hbm.at[pl.ds(r0, chunk_b)], w_v)
                for b in range(chunk_b):
                    pltpu.sync_copy(dst_hbm.at[r0 + b], dst_vs[b])
                for b in range(chunk_b):
                    pltpu.async_copy(
                        xprev.at[src_v.at[b]],
                        rv.at[pl.ds(b * BATCH, BATCH)], sem).wait()

                def group_body(g, carry2):
                    b = g // 8
                    col = (g % 8) * 16
                    wv = w_v[b, pl.ds(col, 16)]
                    e0 = g * 16
                    for j in range(16):
                        e = e0 + j
                        ws = wv[j]
                        rv[e, pl.ds(0, 16)] = rv[e, pl.ds(0, 16)] * ws
                        rv[e, pl.ds(16, 16)] = rv[e, pl.ds(16, 16)] * ws
                    return carry2
                lax.fori_loop(0, chunk_b * 8, group_body, 0)

                for b in range(chunk_b):
                    pltpu.sync_copy(rv.at[pl.ds(b * BATCH, BATCH)],
                                    acc.at[dst_vs[b]], add=True)
                return carry
            lax.fori_loop(0, n_chunks, chunk_body, 0)

        for l in range(N_LAYERS):
            pltpu.sync_copy(z_hbm, acc.at[pl.ds(row0, rows_per_tile)])
            plsc.subcore_barrier()
            edges_phase((x0_hbm, x1_hbm, x2_hbm)[l])
            plsc.subcore_barrier()
            if l < N_LAYERS - 1:
                tgt = (x1_hbm, x2_hbm)[l]
                pltpu.sync_copy(acc.at[pl.ds(row0, rows_per_tile)],
                                tgt.at[pl.ds(hrow0, rows_per_tile)])

        def mean_chunk(k, carry):
            r = pl.multiple_of(row0 + k * mean_rows, 2)
            rh = pl.multiple_of(hrow0 + k * mean_rows, 2)
            pltpu.sync_copy(acc.at[pl.ds(r, mean_rows)], b0)
            pltpu.sync_copy(x0_hbm.at[pl.ds(rh, mean_rows)], b1)
            pltpu.sync_copy(x1_hbm.at[pl.ds(rh, mean_rows)], b2)
            pltpu.sync_copy(x2_hbm.at[pl.ds(rh, mean_rows)], b3)

            def mean_body(t, carry2):
                rr = t // 2
                cc = (t % 2) * 16
                v = (b0[rr, pl.ds(cc, 16)] + b1[rr, pl.ds(cc, 16)]
                     + b2[rr, pl.ds(cc, 16)] + b3[rr, pl.ds(cc, 16)]) * 0.25
                b0[rr, pl.ds(cc, 16)] = v
                return carry2
            lax.fori_loop(0, mean_rows * 2, mean_body, 0)
            pltpu.sync_copy(b0, out_hbm.at[pl.ds(rh, mean_rows)])
            return carry
        lax.fori_loop(0, mean_chunks, mean_chunk, 0)

    @jax.jit
    def run(src_both, dst_r, w_r, x0, z):
        f = pl.kernel(
            body,
            out_type=[
                jax.ShapeDtypeStruct((2 * n_pad, HALF), jnp.float32),
                jax.ShapeDtypeStruct((2 * n_pad, HALF), jnp.float32),
                jax.ShapeDtypeStruct((2 * n_pad, HALF), jnp.float32),
            ],
            mesh=plsc.VectorSubcoreMesh(
                core_axis_name="c", subcore_axis_name="s",
                num_cores=nc, num_subcores=ns),
            compiler_params=pltpu.CompilerParams(use_tc_tiling_on_sc=False),
            scratch_types=(
                [pltpu.VMEM_SHARED((n_pad, HALF), jnp.float32),
                 pltpu.VMEM((chunk_b * BATCH, HALF), jnp.float32),
                 pltpu.VMEM((chunk_b, BATCH), jnp.int32),
                 pltpu.VMEM((chunk_b, BATCH), jnp.float32)]
                + [pltpu.VMEM((BATCH,), jnp.int32) for _ in range(chunk_b)]
                + [pltpu.VMEM((mean_rows, HALF), jnp.float32)
                   for _ in range(4)]
                + [pltpu.SemaphoreType.DMA]),
            interpret=interpret,
        )
        return f(src_both, dst_r, w_r, x0, z)
    return run


_run = _build(N_PAD, NS, NC, EB_PER_TILE, CHUNK_B, MEAN_ROWS)


def kernel(edge_index, edge_weight, user_table, item_table):
    dst = edge_index[0].astype(jnp.int32)
    src = edge_index[1].astype(jnp.int32)
    pad = E_PAD - E
    src_p = jnp.pad(src, (0, pad))
    dst_p = jnp.pad(dst, (0, pad))
    w_p = jnp.pad(edge_weight, (0, pad))
    src_both = jnp.stack([src_p, src_p + N_PAD]).reshape(NC, -1, BATCH)
    dst_r = dst_p.reshape(-1, BATCH)
    w_r = w_p.reshape(-1, BATCH)
    zpad = jnp.zeros((N_PAD - N, HALF), jnp.float32)
    x0 = jnp.concatenate([user_table[:, :HALF], item_table[:, :HALF], zpad,
                          user_table[:, HALF:], item_table[:, HALF:], zpad],
                         axis=0)
    z = jnp.zeros((N_PAD // NS, HALF), jnp.float32)
    out, _x1, _x2 = _run(src_both, dst_r, w_r, x0, z)
    full = jnp.concatenate([out[:N], out[N_PAD:N_PAD + N]], axis=1)
    return (full[:N_USERS], full[N_USERS:])


# speedup vs baseline: 5.2302x; 1.3245x over previous
import jax
import jax.numpy as jnp
from jax import lax
from jax.experimental import pallas as pl
from jax.experimental.pallas import tpu as pltpu
from jax.experimental.pallas import tpu_sc as plsc

N_USERS = 25000
N_ITEMS = 25000
N = N_USERS + N_ITEMS
E = 800000
D = 64
HALF = 32
N_LAYERS = 3

NC = 2
NS = 16
N_PAD = 50048
BATCH = 128
CHUNK_B = 2
NBUF = 3
EB_PER_TILE = 396
E_PAD = NS * EB_PER_TILE * BATCH
MEAN_ROWS = 92
CHUNK_E = CHUNK_B * BATCH


def _build(n_pad, ns, nc, eb_per_tile, mean_rows):
    rows_per_tile = n_pad // ns
    n_chunks = eb_per_tile // CHUNK_B
    n_steps = n_chunks // NBUF
    mean_chunks = rows_per_tile // mean_rows

    def body(*refs):
        (src_hbm, dst_hbm, w_hbm, x0_hbm, z_hbm,
         out_hbm, x1_hbm, x2_hbm, acc) = refs[:9]
        rvs = refs[9:9 + NBUF]
        srcs = refs[9 + NBUF:9 + 2 * NBUF]
        ws = refs[9 + 2 * NBUF:9 + 3 * NBUF]
        dsts = refs[9 + 3 * NBUF:9 + 3 * NBUF + 2 * NBUF]
        semg = refs[9 + 5 * NBUF:9 + 6 * NBUF]
        sems = refs[9 + 6 * NBUF:9 + 7 * NBUF]
        c = lax.axis_index("c")
        s = lax.axis_index("s")
        row0 = pl.multiple_of(s * rows_per_tile, 8)
        erow0 = pl.multiple_of(s * eb_per_tile, 4)
        hrow0 = pl.multiple_of(c * n_pad + s * rows_per_tile, 8)

        def edges_phase(xprev):
            def load_idx(k, gc):
                r0 = erow0 + gc * CHUNK_B
                pltpu.sync_copy(src_hbm.at[c, pl.ds(r0, CHUNK_B)], srcs[k])
                pltpu.sync_copy(w_hbm.at[pl.ds(r0, CHUNK_B)], ws[k])
                pltpu.sync_copy(dst_hbm.at[r0], dsts[2 * k])
                pltpu.sync_copy(dst_hbm.at[r0 + 1], dsts[2 * k + 1])

            def fire_gathers(k):
                for b in range(CHUNK_B):
                    pltpu.async_copy(xprev.at[srcs[k].at[b]],
                                     rvs[k].at[pl.ds(b * BATCH, BATCH)],
                                     semg[k])

            def drain_gathers(k):
                for b in range(CHUNK_B):
                    pltpu.make_async_copy(
                        xprev.at[srcs[k].at[b]],
                        rvs[k].at[pl.ds(b * BATCH, BATCH)],
                        semg[k]).wait()

            def fire_scatter(k):
                for b in range(CHUNK_B):
                    pltpu.async_copy(rvs[k].at[pl.ds(b * BATCH, BATCH)],
                                     acc.at[dsts[2 * k + b]], sems[k],
                                     add=True)

            def drain_scatter(k):
                for b in range(CHUNK_B):
                    pltpu.make_async_copy(
                        rvs[k].at[pl.ds(b * BATCH, BATCH)],
                        acc.at[dsts[2 * k + b]], sems[k]).wait()

            def compute(k):
                rv = rvs[k]
                wk = ws[k]

                def group_body(g, carry2):
                    b = g // 8
                    col = (g % 8) * 16
                    wv = wk[b, pl.ds(col, 16)]
                    e0 = g * 16
                    for j in range(16):
                        e = e0 + j
                        wsc = wv[j]
                        rv[e, pl.ds(0, 16)] = rv[e, pl.ds(0, 16)] * wsc
                        rv[e, pl.ds(16, 16)] = rv[e, pl.ds(16, 16)] * wsc
                    return carry2
                lax.fori_loop(0, CHUNK_B * 8, group_body, 0)

            for k in range(NBUF):
                load_idx(k, k)
                fire_gathers(k)

            def step(t, carry):
                for k in range(NBUF):
                    drain_gathers(k)
                    compute(k)
                    fire_scatter(k)
                    j = (k + NBUF - 1) % NBUF
                    gc = jnp.minimum(NBUF * t + k + 2, n_chunks - 1)
                    if j == NBUF - 1:
                        @pl.when(t > 0)
                        def _():
                            drain_scatter(j)
                            load_idx(j, gc)
                            fire_gathers(j)
                    else:
                        drain_scatter(j)
                        load_idx(j, gc)
                        fire_gathers(j)
                return carry
            lax.fori_loop(0, n_steps, step, 0)
            drain_scatter(NBUF - 1)
            for k in range(NBUF - 1):
                drain_gathers(k)

        for l in range(N_LAYERS):
            pltpu.sync_copy(z_hbm, acc.at[pl.ds(row0, rows_per_tile)])
            plsc.subcore_barrier()
            edges_phase((x0_hbm, x1_hbm, x2_hbm)[l])
            plsc.subcore_barrier()
            if l < N_LAYERS - 1:
                tgt = (x1_hbm, x2_hbm)[l]
                pltpu.sync_copy(acc.at[pl.ds(row0, rows_per_tile)],
                                tgt.at[pl.ds(hrow0, rows_per_tile)])

        b0 = rvs[0].at[pl.ds(0, mean_rows)]
        b1 = rvs[0].at[pl.ds(mean_rows, mean_rows)]
        b2 = rvs[1].at[pl.ds(0, mean_rows)]
        b3 = rvs[1].at[pl.ds(mean_rows, mean_rows)]

        def mean_chunk(kk, carry):
            r = pl.multiple_of(row0 + kk * mean_rows, 2)
            rh = pl.multiple_of(hrow0 + kk * mean_rows, 2)
            pltpu.sync_copy(acc.at[pl.ds(r, mean_rows)], b0)
            pltpu.sync_copy(x0_hbm.at[pl.ds(rh, mean_rows)], b1)
            pltpu.sync_copy(x1_hbm.at[pl.ds(rh, mean_rows)], b2)
            pltpu.sync_copy(x2_hbm.at[pl.ds(rh, mean_rows)], b3)

            def mean_body(t, carry2):
                rr = t // 2
                cc = (t % 2) * 16
                v = (b0[rr, pl.ds(cc, 16)] + b1[rr, pl.ds(cc, 16)]
                     + b2[rr, pl.ds(cc, 16)] + b3[rr, pl.ds(cc, 16)]) * 0.25
                b0[rr, pl.ds(cc, 16)] = v
                return carry2
            lax.fori_loop(0, mean_rows * 2, mean_body, 0)
            pltpu.sync_copy(b0, out_hbm.at[pl.ds(rh, mean_rows)])
            return carry
        lax.fori_loop(0, mean_chunks, mean_chunk, 0)

    @jax.jit
    def run(src_both, dst_r, w_r, x0, z):
        f = pl.kernel(
            body,
            out_type=[
                jax.ShapeDtypeStruct((2 * n_pad, HALF), jnp.float32),
                jax.ShapeDtypeStruct((2 * n_pad, HALF), jnp.float32),
                jax.ShapeDtypeStruct((2 * n_pad, HALF), jnp.float32),
            ],
            mesh=plsc.VectorSubcoreMesh(
                core_axis_name="c", subcore_axis_name="s",
                num_cores=nc, num_subcores=ns),
            compiler_params=pltpu.CompilerParams(use_tc_tiling_on_sc=False),
            scratch_types=(
                [pltpu.VMEM_SHARED((n_pad, HALF), jnp.float32)]
                + [pltpu.VMEM((CHUNK_E, HALF), jnp.float32)
                   for _ in range(NBUF)]
                + [pltpu.VMEM((CHUNK_B, BATCH), jnp.int32)
                   for _ in range(NBUF)]
                + [pltpu.VMEM((CHUNK_B, BATCH), jnp.float32)
                   for _ in range(NBUF)]
                + [pltpu.VMEM((BATCH,), jnp.int32)
                   for _ in range(2 * NBUF)]
                + [pltpu.SemaphoreType.DMA for _ in range(2 * NBUF)]),
        )
        return f(src_both, dst_r, w_r, x0, z)
    return run


_run = _build(N_PAD, NS, NC, EB_PER_TILE, MEAN_ROWS)


def kernel(edge_index, edge_weight, user_table, item_table):
    dst = edge_index[0].astype(jnp.int32)
    src = edge_index[1].astype(jnp.int32)
    pad = E_PAD - E
    src_p = jnp.pad(src, (0, pad))
    dst_p = jnp.pad(dst, (0, pad))
    w_p = jnp.pad(edge_weight, (0, pad))
    src_both = jnp.stack([src_p, src_p + N_PAD]).reshape(NC, -1, BATCH)
    dst_r = dst_p.reshape(-1, BATCH)
    w_r = w_p.reshape(-1, BATCH)
    zpad = jnp.zeros((N_PAD - N, HALF), jnp.float32)
    x0 = jnp.concatenate([user_table[:, :HALF], item_table[:, :HALF], zpad,
                          user_table[:, HALF:], item_table[:, HALF:], zpad],
                         axis=0)
    z = jnp.zeros((N_PAD // NS, HALF), jnp.float32)
    out, _x1, _x2 = _run(src_both, dst_r, w_r, x0, z)
    full = jnp.concatenate([out[:N], out[N_PAD:N_PAD + N]], axis=1)
    return (full[:N_USERS], full[N_USERS:])


# speedup vs baseline: 7.5029x; 1.4345x over previous
import jax
import jax.numpy as jnp
from jax import lax
from jax.experimental import pallas as pl
from jax.experimental.pallas import tpu as pltpu
from jax.experimental.pallas import tpu_sc as plsc

N_USERS = 25000
N_ITEMS = 25000
N = N_USERS + N_ITEMS
E = 800000
D = 64
HALF = 32
N_LAYERS = 3

NC = 2
NS = 16
N_PAD = 50048
BATCH = 128
CHUNK_B = 2
NBUF = 3
EB_PER_TILE = 396
E_PAD = NS * EB_PER_TILE * BATCH
MEAN_ROWS = 92
CHUNK_E = CHUNK_B * BATCH


def _build(n_pad, ns, nc, eb_per_tile, mean_rows):
    rows_per_tile = n_pad // ns
    n_chunks = eb_per_tile // CHUNK_B
    n_steps = n_chunks // NBUF
    mean_chunks = rows_per_tile // mean_rows

    def body(*refs):
        (src_hbm, dst_hbm, w_hbm, x0_hbm, z_hbm,
         out_hbm, x1_hbm, x2_hbm, acc) = refs[:9]
        rvs = refs[9:9 + NBUF]
        srcs = refs[9 + NBUF:9 + 2 * NBUF]
        ws = refs[9 + 2 * NBUF:9 + 3 * NBUF]
        dsts = refs[9 + 3 * NBUF:9 + 5 * NBUF]
        semg = refs[9 + 5 * NBUF:9 + 6 * NBUF]
        sems = refs[9 + 6 * NBUF:9 + 7 * NBUF]
        semi = refs[9 + 7 * NBUF:9 + 8 * NBUF]
        c = lax.axis_index("c")
        s = lax.axis_index("s")
        row0 = pl.multiple_of(s * rows_per_tile, 8)
        erow0 = pl.multiple_of(s * eb_per_tile, 4)
        hrow0 = pl.multiple_of(c * n_pad + s * rows_per_tile, 8)

        def edges_phase(xprev):
            def fire_idx(k, gc):
                r0 = erow0 + gc * CHUNK_B
                pltpu.async_copy(src_hbm.at[c, pl.ds(r0, CHUNK_B)],
                                 srcs[k], semi[k])
                pltpu.async_copy(w_hbm.at[pl.ds(r0, CHUNK_B)],
                                 ws[k], semi[k])
                pltpu.async_copy(dst_hbm.at[r0], dsts[2 * k], semi[k])
                pltpu.async_copy(dst_hbm.at[r0 + 1], dsts[2 * k + 1],
                                 semi[k])

            def drain_idx(k):
                r0 = erow0
                pltpu.make_async_copy(src_hbm.at[c, pl.ds(r0, CHUNK_B)],
                                      srcs[k], semi[k]).wait()
                pltpu.make_async_copy(w_hbm.at[pl.ds(r0, CHUNK_B)],
                                      ws[k], semi[k]).wait()
                pltpu.make_async_copy(dst_hbm.at[r0], dsts[2 * k],
                                      semi[k]).wait()
                pltpu.make_async_copy(dst_hbm.at[r0 + 1], dsts[2 * k + 1],
                                      semi[k]).wait()

            def fire_gathers(k):
                for b in range(CHUNK_B):
                    pltpu.async_copy(xprev.at[srcs[k].at[b]],
                                     rvs[k].at[pl.ds(b * BATCH, BATCH)],
                                     semg[k])

            def drain_gathers(k):
                for b in range(CHUNK_B):
                    pltpu.make_async_copy(
                        xprev.at[srcs[k].at[b]],
                        rvs[k].at[pl.ds(b * BATCH, BATCH)],
                        semg[k]).wait()

            def fire_scatter(k):
                for b in range(CHUNK_B):
                    pltpu.async_copy(rvs[k].at[pl.ds(b * BATCH, BATCH)],
                                     acc.at[dsts[2 * k + b]], sems[k],
                                     add=True)

            def drain_scatter(k):
                for b in range(CHUNK_B):
                    pltpu.make_async_copy(
                        rvs[k].at[pl.ds(b * BATCH, BATCH)],
                        acc.at[dsts[2 * k + b]], sems[k]).wait()

            def compute(k):
                rv = rvs[k]
                wk = ws[k]

                def group_body(g, carry2):
                    b = g // 8
                    col = (g % 8) * 16
                    wv = wk[b, pl.ds(col, 16)]
                    e0 = g * 16
                    for j in range(16):
                        e = e0 + j
                        wsc = wv[j]
                        rv[e, pl.ds(0, 16)] = rv[e, pl.ds(0, 16)] * wsc
                        rv[e, pl.ds(16, 16)] = rv[e, pl.ds(16, 16)] * wsc
                    return carry2
                lax.fori_loop(0, CHUNK_B * 8, group_body, 0)

            fire_idx(0, 0)
            fire_idx(1, 1)
            drain_idx(0)
            fire_gathers(0)
            fire_idx(2, 2)

            def step(t, carry):
                for k in range(NBUF):
                    b1 = (k + 2) % NBUF
                    b2 = (k + 1) % NBUF
                    gc1 = jnp.minimum(NBUF * t + k + 2, n_chunks - 1)
                    if k == 0:
                        @pl.when(t > 0)
                        def _():
                            drain_scatter(b1)
                            fire_idx(b1, gc1)
                    else:
                        drain_scatter(b1)
                        fire_idx(b1, gc1)
                    drain_idx(b2)
                    fire_gathers(b2)
                    drain_gathers(k)
                    compute(k)
                    fire_scatter(k)
                return carry
            lax.fori_loop(0, n_steps, step, 0)
            drain_scatter(NBUF - 1)
            drain_idx(1)
            drain_gathers(0)

        for l in range(N_LAYERS):
            pltpu.sync_copy(z_hbm, acc.at[pl.ds(row0, rows_per_tile)])
            plsc.subcore_barrier()
            edges_phase((x0_hbm, x1_hbm, x2_hbm)[l])
            plsc.subcore_barrier()
            if l < N_LAYERS - 1:
                tgt = (x1_hbm, x2_hbm)[l]
                pltpu.sync_copy(acc.at[pl.ds(row0, rows_per_tile)],
                                tgt.at[pl.ds(hrow0, rows_per_tile)])

        b0 = rvs[0].at[pl.ds(0, mean_rows)]
        b1 = rvs[0].at[pl.ds(mean_rows, mean_rows)]
        b2 = rvs[1].at[pl.ds(0, mean_rows)]
        b3 = rvs[1].at[pl.ds(mean_rows, mean_rows)]

        def mean_chunk(kk, carry):
            r = pl.multiple_of(row0 + kk * mean_rows, 2)
            rh = pl.multiple_of(hrow0 + kk * mean_rows, 2)
            pltpu.sync_copy(acc.at[pl.ds(r, mean_rows)], b0)
            pltpu.sync_copy(x0_hbm.at[pl.ds(rh, mean_rows)], b1)
            pltpu.sync_copy(x1_hbm.at[pl.ds(rh, mean_rows)], b2)
            pltpu.sync_copy(x2_hbm.at[pl.ds(rh, mean_rows)], b3)

            def mean_body(t, carry2):
                rr = t // 2
                cc = (t % 2) * 16
                v = (b0[rr, pl.ds(cc, 16)] + b1[rr, pl.ds(cc, 16)]
                     + b2[rr, pl.ds(cc, 16)] + b3[rr, pl.ds(cc, 16)]) * 0.25
                b0[rr, pl.ds(cc, 16)] = v
                return carry2
            lax.fori_loop(0, mean_rows * 2, mean_body, 0)
            pltpu.sync_copy(b0, out_hbm.at[pl.ds(rh, mean_rows)])
            return carry
        lax.fori_loop(0, mean_chunks, mean_chunk, 0)

    @jax.jit
    def run(src_both, dst_r, w_r, x0, z):
        f = pl.kernel(
            body,
            out_type=[
                jax.ShapeDtypeStruct((2 * n_pad, HALF), jnp.float32),
                jax.ShapeDtypeStruct((2 * n_pad, HALF), jnp.float32),
                jax.ShapeDtypeStruct((2 * n_pad, HALF), jnp.float32),
            ],
            mesh=plsc.VectorSubcoreMesh(
                core_axis_name="c", subcore_axis_name="s",
                num_cores=nc, num_subcores=ns),
            compiler_params=pltpu.CompilerParams(use_tc_tiling_on_sc=False),
            scratch_types=(
                [pltpu.VMEM_SHARED((n_pad, HALF), jnp.float32)]
                + [pltpu.VMEM((CHUNK_E, HALF), jnp.float32)
                   for _ in range(NBUF)]
                + [pltpu.VMEM((CHUNK_B, BATCH), jnp.int32)
                   for _ in range(NBUF)]
                + [pltpu.VMEM((CHUNK_B, BATCH), jnp.float32)
                   for _ in range(NBUF)]
                + [pltpu.VMEM((BATCH,), jnp.int32)
                   for _ in range(2 * NBUF)]
                + [pltpu.SemaphoreType.DMA for _ in range(3 * NBUF)]),
        )
        return f(src_both, dst_r, w_r, x0, z)
    return run


_run = _build(N_PAD, NS, NC, EB_PER_TILE, MEAN_ROWS)


def kernel(edge_index, edge_weight, user_table, item_table):
    dst = edge_index[0].astype(jnp.int32)
    src = edge_index[1].astype(jnp.int32)
    pad = E_PAD - E
    src_p = jnp.pad(src, (0, pad))
    dst_p = jnp.pad(dst, (0, pad))
    w_p = jnp.pad(edge_weight, (0, pad))
    src_both = jnp.stack([src_p, src_p + N_PAD]).reshape(NC, -1, BATCH)
    dst_r = dst_p.reshape(-1, BATCH)
    w_r = w_p.reshape(-1, BATCH)
    zpad = jnp.zeros((N_PAD - N, HALF), jnp.float32)
    x0 = jnp.concatenate([user_table[:, :HALF], item_table[:, :HALF], zpad,
                          user_table[:, HALF:], item_table[:, HALF:], zpad],
                         axis=0)
    z = jnp.zeros((N_PAD // NS, HALF), jnp.float32)
    out, _x1, _x2 = _run(src_both, dst_r, w_r, x0, z)
    full = jnp.concatenate([out[:N], out[N_PAD:N_PAD + N]], axis=1)
    return (full[:N_USERS], full[N_USERS:])


# speedup vs baseline: 7.8234x; 1.0427x over previous
import jax
import jax.numpy as jnp
from jax import lax
from jax.experimental import pallas as pl
from jax.experimental.pallas import tpu as pltpu
from jax.experimental.pallas import tpu_sc as plsc

N_USERS = 25000
N_ITEMS = 25000
N = N_USERS + N_ITEMS
E = 800000
D = 64
HALF = 32
N_LAYERS = 3

NC = 2
NS = 16
N_PAD = 50048
BATCH = 128
CHUNK_B = 2
NBUF = 3
EB_PER_TILE = 396
E_PAD = NS * EB_PER_TILE * BATCH
MEAN_ROWS = 92
CHUNK_E = CHUNK_B * BATCH


def _build(n_pad, ns, nc, eb_per_tile, mean_rows):
    rows_per_tile = n_pad // ns
    n_chunks = eb_per_tile // CHUNK_B
    n_steps = n_chunks // NBUF
    mean_chunks = rows_per_tile // mean_rows

    def body(*refs):
        (ed_hbm, x0_hbm, z_hbm,
         out_hbm, x1_hbm, x2_hbm, acc) = refs[:7]
        rvs = refs[7:7 + NBUF]
        edi = refs[7 + NBUF:7 + 2 * NBUF]
        semg = refs[7 + 2 * NBUF:7 + 3 * NBUF]
        sems = refs[7 + 3 * NBUF:7 + 4 * NBUF]
        semi = refs[7 + 4 * NBUF:7 + 5 * NBUF]
        c = lax.axis_index("c")
        s = lax.axis_index("s")
        row0 = pl.multiple_of(s * rows_per_tile, 8)
        erow0 = pl.multiple_of(s * eb_per_tile, 4)
        hrow0 = pl.multiple_of(c * n_pad + s * rows_per_tile, 8)

        def edges_phase(xprev):
            def fire_idx(k, gc):
                r0 = erow0 + gc * CHUNK_B
                pltpu.async_copy(ed_hbm.at[c, pl.ds(r0, CHUNK_B)],
                                 edi[k], semi[k])

            def drain_idx(k):
                pltpu.make_async_copy(ed_hbm.at[c, pl.ds(erow0, CHUNK_B)],
                                      edi[k], semi[k]).wait()

            def fire_gathers(k):
                for b in range(CHUNK_B):
                    pltpu.async_copy(xprev.at[edi[k].at[b, 0]],
                                     rvs[k].at[pl.ds(b * BATCH, BATCH)],
                                     semg[k])

            def drain_gathers(k):
                for b in range(CHUNK_B):
                    pltpu.make_async_copy(
                        xprev.at[edi[k].at[b, 0]],
                        rvs[k].at[pl.ds(b * BATCH, BATCH)],
                        semg[k]).wait()

            def fire_scatter(k):
                for b in range(CHUNK_B):
                    pltpu.async_copy(rvs[k].at[pl.ds(b * BATCH, BATCH)],
                                     acc.at[edi[k].at[b, 2]], sems[k],
                                     add=True)

            def drain_scatter(k):
                for b in range(CHUNK_B):
                    pltpu.make_async_copy(
                        rvs[k].at[pl.ds(b * BATCH, BATCH)],
                        acc.at[edi[k].at[b, 2]], sems[k]).wait()

            def compute(k):
                rv = rvs[k]
                wk = edi[k]

                def group_body(g, carry2):
                    b = g // 8
                    col = (g % 8) * 16
                    wv = plsc.bitcast(wk[b, 1, pl.ds(col, 16)], jnp.float32)
                    e0 = g * 16
                    for j in range(16):
                        e = e0 + j
                        wsc = wv[j]
                        rv[e, pl.ds(0, 16)] = rv[e, pl.ds(0, 16)] * wsc
                        rv[e, pl.ds(16, 16)] = rv[e, pl.ds(16, 16)] * wsc
                    return carry2
                lax.fori_loop(0, CHUNK_B * 8, group_body, 0)

            fire_idx(0, 0)
            fire_idx(1, 1)
            drain_idx(0)
            fire_gathers(0)
            fire_idx(2, 2)

            def step(t, carry):
                for k in range(NBUF):
                    b1 = (k + 2) % NBUF
                    b2 = (k + 1) % NBUF
                    gc1 = jnp.minimum(NBUF * t + k + 2, n_chunks - 1)
                    if k == 0:
                        @pl.when(t > 0)
                        def _():
                            drain_scatter(b1)
                            fire_idx(b1, gc1)
                    else:
                        drain_scatter(b1)
                        fire_idx(b1, gc1)
                    drain_idx(b2)
                    fire_gathers(b2)
                    drain_gathers(k)
                    compute(k)
                    fire_scatter(k)
                return carry
            lax.fori_loop(0, n_steps, step, 0)
            drain_scatter(NBUF - 1)
            drain_idx(1)
            drain_gathers(0)

        for l in range(N_LAYERS):
            pltpu.sync_copy(z_hbm, acc.at[pl.ds(row0, rows_per_tile)])
            plsc.subcore_barrier()
            edges_phase((x0_hbm, x1_hbm, x2_hbm)[l])
            plsc.subcore_barrier()
            if l < N_LAYERS - 1:
                tgt = (x1_hbm, x2_hbm)[l]
                pltpu.sync_copy(acc.at[pl.ds(row0, rows_per_tile)],
                                tgt.at[pl.ds(hrow0, rows_per_tile)])

        b0 = rvs[0].at[pl.ds(0, mean_rows)]
        b1 = rvs[0].at[pl.ds(mean_rows, mean_rows)]
        b2 = rvs[1].at[pl.ds(0, mean_rows)]
        b3 = rvs[1].at[pl.ds(mean_rows, mean_rows)]

        def mean_chunk(kk, carry):
            r = pl.multiple_of(row0 + kk * mean_rows, 2)
            rh = pl.multiple_of(hrow0 + kk * mean_rows, 2)
            pltpu.sync_copy(acc.at[pl.ds(r, mean_rows)], b0)
            pltpu.sync_copy(x0_hbm.at[pl.ds(rh, mean_rows)], b1)
            pltpu.sync_copy(x1_hbm.at[pl.ds(rh, mean_rows)], b2)
            pltpu.sync_copy(x2_hbm.at[pl.ds(rh, mean_rows)], b3)

            def mean_body(t, carry2):
                rr = t // 2
                cc = (t % 2) * 16
                v = (b0[rr, pl.ds(cc, 16)] + b1[rr, pl.ds(cc, 16)]
                     + b2[rr, pl.ds(cc, 16)] + b3[rr, pl.ds(cc, 16)]) * 0.25
                b0[rr, pl.ds(cc, 16)] = v
                return carry2
            lax.fori_loop(0, mean_rows * 2, mean_body, 0)
            pltpu.sync_copy(b0, out_hbm.at[pl.ds(rh, mean_rows)])
            return carry
        lax.fori_loop(0, mean_chunks, mean_chunk, 0)

    @jax.jit
    def run(ed, x0, z):
        f = pl.kernel(
            body,
            out_type=[
                jax.ShapeDtypeStruct((2 * n_pad, HALF), jnp.float32),
                jax.ShapeDtypeStruct((2 * n_pad, HALF), jnp.float32),
                jax.ShapeDtypeStruct((2 * n_pad, HALF), jnp.float32),
            ],
            mesh=plsc.VectorSubcoreMesh(
                core_axis_name="c", subcore_axis_name="s",
                num_cores=nc, num_subcores=ns),
            compiler_params=pltpu.CompilerParams(
                use_tc_tiling_on_sc=False, needs_layout_passes=False),
            scratch_types=(
                [pltpu.VMEM_SHARED((n_pad, HALF), jnp.float32)]
                + [pltpu.VMEM((CHUNK_E, HALF), jnp.float32)
                   for _ in range(NBUF)]
                + [pltpu.VMEM((CHUNK_B, 3, BATCH), jnp.int32)
                   for _ in range(NBUF)]
                + [pltpu.SemaphoreType.DMA for _ in range(3 * NBUF)]),
        )
        return f(ed, x0, z)
    return run


_run = _build(N_PAD, NS, NC, EB_PER_TILE, MEAN_ROWS)


def kernel(edge_index, edge_weight, user_table, item_table):
    dst = edge_index[0].astype(jnp.int32)
    src = edge_index[1].astype(jnp.int32)
    pad = E_PAD - E
    src_p = jnp.pad(src, (0, pad))
    dst_p = jnp.pad(dst, (0, pad))
    w_p = jnp.pad(edge_weight, (0, pad))
    w_bits = lax.bitcast_convert_type(w_p, jnp.int32)
    ed = jnp.stack([
        jnp.stack([(src_p + cc * N_PAD).reshape(-1, BATCH),
                   w_bits.reshape(-1, BATCH),
                   dst_p.reshape(-1, BATCH)], axis=1)
        for cc in range(NC)], axis=0)
    zpad = jnp.zeros((N_PAD - N, HALF), jnp.float32)
    x0 = jnp.concatenate([user_table[:, :HALF], item_table[:, :HALF], zpad,
                          user_table[:, HALF:], item_table[:, HALF:], zpad],
                         axis=0)
    z = jnp.zeros((N_PAD // NS, HALF), jnp.float32)
    out, _x1, _x2 = _run(ed, x0, z)
    full = jnp.concatenate([out[:N], out[N_PAD:N_PAD + N]], axis=1)
    return (full[:N_USERS], full[N_USERS:])


# speedup vs baseline: 8.0868x; 1.0337x over previous
import jax
import jax.numpy as jnp
from jax import lax
from jax.experimental import pallas as pl
from jax.experimental.pallas import tpu as pltpu
from jax.experimental.pallas import tpu_sc as plsc

N_USERS = 25000
N_ITEMS = 25000
N = N_USERS + N_ITEMS
E = 800000
D = 64
HALF = 32
N_LAYERS = 3

NC = 2
NS = 16
N_PAD = 50048
BATCH = 128
CHUNK_B = 2
NBUF = 3
EB_PER_TILE = 396
E_PAD = NS * EB_PER_TILE * BATCH
MEAN_ROWS = 92
CHUNK_E = CHUNK_B * BATCH


def _build(n_pad, ns, nc, eb_per_tile, mean_rows):
    rows_per_tile = n_pad // ns
    n_chunks = eb_per_tile // CHUNK_B
    n_steps = n_chunks // NBUF
    mean_chunks = rows_per_tile // mean_rows

    def body(*refs):
        (ed_hbm, x0_hbm, z_hbm,
         out_hbm, x1_hbm, x2_hbm, acc) = refs[:7]
        rvs = refs[7:7 + NBUF]
        edi = refs[7 + NBUF:7 + 2 * NBUF]
        semg = refs[7 + 2 * NBUF:7 + 3 * NBUF]
        sems = refs[7 + 3 * NBUF:7 + 4 * NBUF]
        semi = refs[7 + 4 * NBUF:7 + 5 * NBUF]
        c = lax.axis_index("c")
        s = lax.axis_index("s")
        row0 = pl.multiple_of(s * rows_per_tile, 8)
        echunk0 = s * n_chunks
        hrow0 = pl.multiple_of(c * n_pad + s * rows_per_tile, 8)

        def edges_phase(xprev):
            def fire_idx(k, gc):
                pltpu.async_copy(ed_hbm.at[c, echunk0 + gc],
                                 edi[k], semi[k])

            def drain_idx(k):
                pltpu.make_async_copy(ed_hbm.at[c, echunk0],
                                      edi[k], semi[k]).wait()

            def fire_gathers(k):
                pltpu.async_copy(xprev.at[edi[k].at[0]], rvs[k], semg[k])

            def drain_gathers(k):
                pltpu.make_async_copy(xprev.at[edi[k].at[0]],
                                      rvs[k], semg[k]).wait()

            def fire_scatter(k):
                pltpu.async_copy(rvs[k], acc.at[edi[k].at[2]], sems[k],
                                 add=True)

            def drain_scatter(k):
                pltpu.make_async_copy(rvs[k], acc.at[edi[k].at[2]],
                                      sems[k]).wait()

            def compute(k):
                rv = rvs[k]
                wk = edi[k]

                def group_body(g, carry2):
                    col = g * 16
                    wv = plsc.bitcast(wk[1, pl.ds(col, 16)], jnp.float32)
                    for j in range(16):
                        e = col + j
                        wsc = wv[j]
                        rv[e, pl.ds(0, 16)] = rv[e, pl.ds(0, 16)] * wsc
                        rv[e, pl.ds(16, 16)] = rv[e, pl.ds(16, 16)] * wsc
                    return carry2
                lax.fori_loop(0, CHUNK_B * 8, group_body, 0)

            fire_idx(0, 0)
            fire_idx(1, 1)
            drain_idx(0)
            fire_gathers(0)
            fire_idx(2, 2)

            def step(t, carry):
                for k in range(NBUF):
                    b1 = (k + 2) % NBUF
                    b2 = (k + 1) % NBUF
                    gc1 = jnp.minimum(NBUF * t + k + 2, n_chunks - 1)
                    if k == 0:
                        @pl.when(t > 0)
                        def _():
                            drain_scatter(b1)
                            fire_idx(b1, gc1)
                    else:
                        drain_scatter(b1)
                        fire_idx(b1, gc1)
                    drain_idx(b2)
                    fire_gathers(b2)
                    drain_gathers(k)
                    compute(k)
                    fire_scatter(k)
                return carry
            lax.fori_loop(0, n_steps, step, 0)
            drain_scatter(NBUF - 1)
            drain_idx(1)
            drain_gathers(0)

        for l in range(N_LAYERS):
            pltpu.sync_copy(z_hbm, acc.at[pl.ds(row0, rows_per_tile)])
            plsc.subcore_barrier()
            edges_phase((x0_hbm, x1_hbm, x2_hbm)[l])
            plsc.subcore_barrier()
            if l < N_LAYERS - 1:
                tgt = (x1_hbm, x2_hbm)[l]
                pltpu.sync_copy(acc.at[pl.ds(row0, rows_per_tile)],
                                tgt.at[pl.ds(hrow0, rows_per_tile)])

        b0 = rvs[0].at[pl.ds(0, mean_rows)]
        b1 = rvs[0].at[pl.ds(mean_rows, mean_rows)]
        b2 = rvs[1].at[pl.ds(0, mean_rows)]
        b3 = rvs[1].at[pl.ds(mean_rows, mean_rows)]

        def mean_chunk(kk, carry):
            r = pl.multiple_of(row0 + kk * mean_rows, 2)
            rh = pl.multiple_of(hrow0 + kk * mean_rows, 2)
            h1 = pltpu.async_copy(x0_hbm.at[pl.ds(rh, mean_rows)], b1,
                                  semg[0])
            h2 = pltpu.async_copy(x1_hbm.at[pl.ds(rh, mean_rows)], b2,
                                  semg[1])
            h3 = pltpu.async_copy(x2_hbm.at[pl.ds(rh, mean_rows)], b3,
                                  semg[2])
            pltpu.sync_copy(acc.at[pl.ds(r, mean_rows)], b0)
            h1.wait()
            h2.wait()
            h3.wait()

            def mean_body(t, carry2):
                rr = t // 2
                cc = (t % 2) * 16
                v = (b0[rr, pl.ds(cc, 16)] + b1[rr, pl.ds(cc, 16)]
                     + b2[rr, pl.ds(cc, 16)] + b3[rr, pl.ds(cc, 16)]) * 0.25
                b0[rr, pl.ds(cc, 16)] = v
                return carry2
            lax.fori_loop(0, mean_rows * 2, mean_body, 0)
            pltpu.sync_copy(b0, out_hbm.at[pl.ds(rh, mean_rows)])
            return carry
        lax.fori_loop(0, mean_chunks, mean_chunk, 0)

    @jax.jit
    def run(ed, x0, z):
        f = pl.kernel(
            body,
            out_type=[
                jax.ShapeDtypeStruct((2 * n_pad, HALF), jnp.float32),
                jax.ShapeDtypeStruct((2 * n_pad, HALF), jnp.float32),
                jax.ShapeDtypeStruct((2 * n_pad, HALF), jnp.float32),
            ],
            mesh=plsc.VectorSubcoreMesh(
                core_axis_name="c", subcore_axis_name="s",
                num_cores=nc, num_subcores=ns),
            compiler_params=pltpu.CompilerParams(
                use_tc_tiling_on_sc=False, needs_layout_passes=False),
            scratch_types=(
                [pltpu.VMEM_SHARED((n_pad, HALF), jnp.float32)]
                + [pltpu.VMEM((CHUNK_E, HALF), jnp.float32)
                   for _ in range(NBUF)]
                + [pltpu.VMEM((3, CHUNK_E), jnp.int32)
                   for _ in range(NBUF)]
                + [pltpu.SemaphoreType.DMA for _ in range(3 * NBUF)]),
        )
        return f(ed, x0, z)
    return run


_run = _build(N_PAD, NS, NC, EB_PER_TILE, MEAN_ROWS)


def kernel(edge_index, edge_weight, user_table, item_table):
    dst = edge_index[0].astype(jnp.int32)
    src = edge_index[1].astype(jnp.int32)
    pad = E_PAD - E
    src_p = jnp.pad(src, (0, pad))
    dst_p = jnp.pad(dst, (0, pad))
    w_p = jnp.pad(edge_weight, (0, pad))
    w_bits = lax.bitcast_convert_type(w_p, jnp.int32)
    ed = jnp.stack([
        jnp.stack([(src_p + cc * N_PAD).reshape(-1, CHUNK_E),
                   w_bits.reshape(-1, CHUNK_E),
                   dst_p.reshape(-1, CHUNK_E)], axis=1)
        for cc in range(NC)], axis=0)
    zpad = jnp.zeros((N_PAD - N, HALF), jnp.float32)
    x0 = jnp.concatenate([user_table[:, :HALF], item_table[:, :HALF], zpad,
                          user_table[:, HALF:], item_table[:, HALF:], zpad],
                         axis=0)
    z = jnp.zeros((N_PAD // NS, HALF), jnp.float32)
    out, _x1, _x2 = _run(ed, x0, z)
    full = jnp.concatenate([out[:N], out[N_PAD:N_PAD + N]], axis=1)
    return (full[:N_USERS], full[N_USERS:])
